# Initial kernel scaffold; baseline (speedup 1.0000x reference)
import jax
import jax.numpy as jnp
from jax.experimental import pallas as pl


def kernel(x, edge_index, W1l, b1l, W1r, W2l, b2l, W2r):
    raise NotImplementedError("write your pallas kernel here")


# speedup vs baseline: 6.7780x; 6.7780x over previous
import functools

import jax
import jax.numpy as jnp
from jax import lax
from jax.experimental import pallas as pl
from jax.experimental.pallas import tpu as pltpu
from jax.experimental.pallas import tpu_sc as plsc

N_NODES = 10000
D = 128
NC, NS = 2, 16
NW = NC * NS
CHUNK = 128
CPT = 80
EPT = CHUNK * CPT
E_PAD = EPT * NW
N_PAD = 10240
ROWS_PER_TILE = N_PAD // NS
DEGW = 8
ZROWS = 64


def _agg_body(with_deg, *args):
    if with_deg:
        (src_hbm, dst_hbm, table_hbm,
         agg_out, deg_out, src_idx, dst_idx, rows, ones, zbuf, zd, gsem,
         agg_sh, deg_sh) = args
    else:
        (src_hbm, dst_hbm, table_hbm,
         agg_out, src_idx, dst_idx, rows, zbuf, gsem, agg_sh) = args
    cid = lax.axis_index("c")
    sid = lax.axis_index("s")
    wid = sid * NC + cid

    z16 = jnp.zeros((16,), jnp.float32)

    def zero_zbuf(r, c):
        for k in range(D // 16):
            zbuf[r, pl.ds(k * 16, 16)] = z16
        return c
    lax.fori_loop(0, ZROWS, zero_zbuf, 0)

    def zero_agg(t, c):
        pltpu.sync_copy(zbuf, agg_sh.at[pl.ds(sid * ROWS_PER_TILE + t * ZROWS, ZROWS)])
        return c
    lax.fori_loop(0, ROWS_PER_TILE // ZROWS, zero_agg, 0)

    if with_deg:
        one16 = jnp.ones((16,), jnp.float32)
        for k in range(CHUNK // 16):
            ones[pl.ds(k * 16, 16)] = one16
        for k in range(ROWS_PER_TILE // 16):
            zd[pl.ds(k * 16, 16)] = z16
        pltpu.sync_copy(zd, deg_sh.at[pl.ds(sid * ROWS_PER_TILE, ROWS_PER_TILE)])

    plsc.subcore_barrier()

    def step(t, c):
        base = wid * EPT + t * CHUNK
        pltpu.sync_copy(src_hbm.at[pl.ds(base, CHUNK)], src_idx)
        pltpu.sync_copy(dst_hbm.at[pl.ds(base, CHUNK)], dst_idx)
        pltpu.async_copy(table_hbm.at[src_idx], rows, gsem).wait()
        pltpu.sync_copy(rows, agg_sh.at[dst_idx], add=True)
        if with_deg:
            pltpu.sync_copy(ones, deg_sh.at[dst_idx], add=True)
        return c
    lax.fori_loop(0, CPT, step, 0)

    plsc.subcore_barrier()

    pltpu.sync_copy(agg_sh.at[pl.ds(sid * ROWS_PER_TILE, ROWS_PER_TILE)],
                    agg_out.at[cid, pl.ds(sid * ROWS_PER_TILE, ROWS_PER_TILE)])
    if with_deg:
        pltpu.sync_copy(deg_sh.at[pl.ds(sid * ROWS_PER_TILE, ROWS_PER_TILE)],
                        deg_out.at[pl.ds(cid * N_PAD + sid * ROWS_PER_TILE,
                                         ROWS_PER_TILE)])


def _make_agg(with_deg):
    mesh = plsc.VectorSubcoreMesh(core_axis_name="c", subcore_axis_name="s",
                                  num_cores=NC, num_subcores=NS)
    outs = [jax.ShapeDtypeStruct((NC, N_PAD, D), jnp.float32)]
    scratch = [
        pltpu.VMEM((CHUNK,), jnp.int32),
        pltpu.VMEM((CHUNK,), jnp.int32),
        pltpu.VMEM((CHUNK, D), jnp.float32),
    ]
    if with_deg:
        outs.append(jax.ShapeDtypeStruct((NC * N_PAD,), jnp.float32))
        scratch.append(pltpu.VMEM((CHUNK,), jnp.float32))
    scratch.append(pltpu.VMEM((ZROWS, D), jnp.float32))
    if with_deg:
        scratch.append(pltpu.VMEM((ROWS_PER_TILE,), jnp.float32))
    scratch.append(pltpu.SemaphoreType.DMA)
    scratch.append(pltpu.VMEM_SHARED((N_PAD, D), jnp.float32))
    if with_deg:
        scratch.append(pltpu.VMEM_SHARED((N_PAD,), jnp.float32))
    return pl.kernel(
        functools.partial(_agg_body, with_deg),
        out_type=tuple(outs) if with_deg else outs[0],
        mesh=mesh,
        scratch_types=scratch,
    )


_agg_with_deg = _make_agg(True)
_agg_only = _make_agg(False)


def _layer_body(do_relu, aggp_ref, degp_ref, x_ref, wl_ref, bl_ref, wr_ref, out_ref):
    deg = degp_ref[0] + degp_ref[1]
    inv = 1.0 / jnp.maximum(deg, 1.0)
    mean = (aggp_ref[0] + aggp_ref[1]) * inv
    dn = (((1,), (1,)), ((), ()))
    acc = lax.dot_general(mean, wl_ref[...], dn, preferred_element_type=jnp.float32)
    acc = acc + bl_ref[...]
    acc = acc + lax.dot_general(x_ref[...], wr_ref[...], dn,
                                preferred_element_type=jnp.float32)
    out_ref[...] = jnp.maximum(acc, 0.0) if do_relu else acc


def _tc_layer(aggp, degp, x, Wl, bl, Wr, do_relu):
    BLK = 2000
    grid = (N_NODES // BLK,)
    return pl.pallas_call(
        functools.partial(_layer_body, do_relu),
        grid=grid,
        in_specs=[
            pl.BlockSpec((NC, BLK, D), lambda i: (0, i, 0)),
            pl.BlockSpec((NC, BLK, 1), lambda i: (0, i, 0)),
            pl.BlockSpec((BLK, D), lambda i: (i, 0)),
            pl.BlockSpec((D, D), lambda i: (0, 0)),
            pl.BlockSpec((1, D), lambda i: (0, 0)),
            pl.BlockSpec((D, D), lambda i: (0, 0)),
        ],
        out_specs=pl.BlockSpec((BLK, D), lambda i: (i, 0)),
        out_shape=jax.ShapeDtypeStruct((N_NODES, D), jnp.float32),
    )(aggp, degp, x, Wl, bl.reshape(1, D), Wr)


@jax.jit
def kernel(x, edge_index, W1l, b1l, W1r, W2l, b2l, W2r):
    src = edge_index[0].astype(jnp.int32)
    dst = edge_index[1].astype(jnp.int32)
    pad = E_PAD - src.shape[0]
    pi = jnp.arange(pad, dtype=jnp.int32)
    src_p = jnp.concatenate([src, pi % N_NODES])
    dst_p = jnp.concatenate([dst, N_NODES + pi % (N_PAD - N_NODES)])

    agg1, deg_flat = _agg_with_deg(src_p, dst_p, x)
    degp = deg_flat.reshape(NC, N_PAD, 1)
    h = _tc_layer(agg1, degp, x, W1l, b1l, W1r, True)
    agg2 = _agg_only(src_p, dst_p, h)
    return _tc_layer(agg2, degp, h, W2l, b2l, W2r, False)


# speedup vs baseline: 13.6719x; 2.0171x over previous
import functools

import jax
import jax.numpy as jnp
from jax import lax
from jax.experimental import pallas as pl
from jax.experimental.pallas import tpu as pltpu
from jax.experimental.pallas import tpu_sc as plsc

N_NODES = 10000
D = 128
NC, NS = 2, 16
NW = NC * NS
CHUNK = 128
CPT = 80
EPT = CHUNK * CPT
E_PAD = EPT * NW
N_PAD = 10240
ROWS_PER_TILE = N_PAD // NS
ZROWS = 16


def _agg_body(with_deg, *args):
    if with_deg:
        (edges_hbm, table_hbm,
         agg_out, deg_out,
         ib0, ib1, ib2, ib3, rows0, rows1, ones, zbuf, zd,
         isem0, isem1, isem2, isem3, gsem0, gsem1, ssem0, ssem1,
         dsem0, dsem1,
         agg_sh, deg_sh) = args
    else:
        (edges_hbm, table_hbm,
         agg_out,
         ib0, ib1, ib2, ib3, rows0, rows1, zbuf,
         isem0, isem1, isem2, isem3, gsem0, gsem1, ssem0, ssem1,
         agg_sh) = args
    ib = (ib0, ib1, ib2, ib3)
    isem = (isem0, isem1, isem2, isem3)
    rows = (rows0, rows1)
    gsem = (gsem0, gsem1)
    ssem = (ssem0, ssem1)
    cid = lax.axis_index("c")
    sid = lax.axis_index("s")
    wid = sid * NC + cid

    for k in range(3):
        pltpu.async_copy(edges_hbm.at[wid, k], ib[k], isem[k])
    pltpu.make_async_copy(edges_hbm.at[wid, 0], ib0, isem0).wait()
    pltpu.async_copy(table_hbm.at[ib0.at[0]], rows0, gsem0)

    z16 = jnp.zeros((16,), jnp.float32)

    def zero_zbuf(r, c):
        for k in range(D // 16):
            zbuf[r, pl.ds(k * 16, 16)] = z16
        return c
    lax.fori_loop(0, ZROWS, zero_zbuf, 0)

    def zero_agg(t, c):
        pltpu.sync_copy(zbuf, agg_sh.at[pl.ds(sid * ROWS_PER_TILE + t * ZROWS, ZROWS)])
        return c
    lax.fori_loop(0, ROWS_PER_TILE // ZROWS, zero_agg, 0)

    if with_deg:
        one16 = jnp.ones((16,), jnp.float32)
        for k in range(CHUNK // 16):
            ones[pl.ds(k * 16, 16)] = one16
        for k in range(ROWS_PER_TILE // 16):
            zd[pl.ds(k * 16, 16)] = z16
        pltpu.sync_copy(zd, deg_sh.at[pl.ds(sid * ROWS_PER_TILE, ROWS_PER_TILE)])

    plsc.subcore_barrier()

    dsem = (dsem0, dsem1) if with_deg else None

    def step(g, c):
        for b in (0, 1, 2, 3):
            t = 4 * g + b
            rb = b % 2
            nrb = 1 - rb
            pb = (b + 3) % 4

            @pl.when(t >= 1)
            def _wait_prev_scatter():
                pltpu.make_async_copy(
                    rows[nrb], agg_sh.at[ib[pb].at[1]], ssem[nrb]).wait()
                if with_deg:
                    pltpu.make_async_copy(
                        ones, deg_sh.at[ib[pb].at[1]], dsem[nrb]).wait()

            @pl.when(t + 3 < CPT)
            def _prefetch_idx():
                pltpu.async_copy(edges_hbm.at[wid, t + 3], ib[pb], isem[pb])

            @pl.when(t + 1 < CPT)
            def _issue_next_gather():
                nib = ib[(b + 1) % 4]
                pltpu.make_async_copy(
                    edges_hbm.at[wid, t + 1], nib, isem[(b + 1) % 4]).wait()
                pltpu.async_copy(table_hbm.at[nib.at[0]], rows[nrb], gsem[nrb])

            pltpu.make_async_copy(
                table_hbm.at[ib[b].at[0]], rows[rb], gsem[rb]).wait()
            pltpu.async_copy(
                rows[rb], agg_sh.at[ib[b].at[1]], ssem[rb], add=True)
            if with_deg:
                pltpu.async_copy(ones, deg_sh.at[ib[b].at[1]], dsem[rb], add=True)
        return c
    lax.fori_loop(0, CPT // 4, step, 0)

    pltpu.make_async_copy(rows[1], agg_sh.at[ib[3].at[1]], ssem[1]).wait()
    if with_deg:
        pltpu.make_async_copy(ones, deg_sh.at[ib[3].at[1]], dsem[1]).wait()

    plsc.subcore_barrier()

    pltpu.sync_copy(agg_sh.at[pl.ds(sid * ROWS_PER_TILE, ROWS_PER_TILE)],
                    agg_out.at[cid, pl.ds(sid * ROWS_PER_TILE, ROWS_PER_TILE)])
    if with_deg:
        pltpu.sync_copy(deg_sh.at[pl.ds(sid * ROWS_PER_TILE, ROWS_PER_TILE)],
                        deg_out.at[pl.ds(cid * N_PAD + sid * ROWS_PER_TILE,
                                         ROWS_PER_TILE)])


def _make_agg(with_deg):
    mesh = plsc.VectorSubcoreMesh(core_axis_name="c", subcore_axis_name="s",
                                  num_cores=NC, num_subcores=NS)
    outs = [jax.ShapeDtypeStruct((NC, N_PAD, D), jnp.float32)]
    if with_deg:
        outs.append(jax.ShapeDtypeStruct((NC * N_PAD,), jnp.float32))
    scratch = [
        pltpu.VMEM((2, CHUNK), jnp.int32),
        pltpu.VMEM((2, CHUNK), jnp.int32),
        pltpu.VMEM((2, CHUNK), jnp.int32),
        pltpu.VMEM((2, CHUNK), jnp.int32),
        pltpu.VMEM((CHUNK, D), jnp.float32),
        pltpu.VMEM((CHUNK, D), jnp.float32),
    ]
    if with_deg:
        scratch.append(pltpu.VMEM((CHUNK,), jnp.float32))
    scratch.append(pltpu.VMEM((ZROWS, D), jnp.float32))
    if with_deg:
        scratch.append(pltpu.VMEM((ROWS_PER_TILE,), jnp.float32))
    scratch += [pltpu.SemaphoreType.DMA] * (10 if with_deg else 8)
    scratch.append(pltpu.VMEM_SHARED((N_PAD, D), jnp.float32))
    if with_deg:
        scratch.append(pltpu.VMEM_SHARED((N_PAD,), jnp.float32))
    return pl.kernel(
        functools.partial(_agg_body, with_deg),
        out_type=tuple(outs) if with_deg else outs[0],
        mesh=mesh,
        scratch_types=scratch,
    )


_agg_with_deg = _make_agg(True)
_agg_only = _make_agg(False)


def _layer_body(do_relu, aggp_ref, degp_ref, x_ref, wl_ref, bl_ref, wr_ref, out_ref):
    deg = degp_ref[0] + degp_ref[1]
    inv = 1.0 / jnp.maximum(deg, 1.0)
    mean = (aggp_ref[0] + aggp_ref[1]) * inv
    dn = (((1,), (1,)), ((), ()))
    acc = lax.dot_general(mean, wl_ref[...], dn, preferred_element_type=jnp.float32)
    acc = acc + bl_ref[...]
    acc = acc + lax.dot_general(x_ref[...], wr_ref[...], dn,
                                preferred_element_type=jnp.float32)
    out_ref[...] = jnp.maximum(acc, 0.0) if do_relu else acc


def _tc_layer(aggp, degp, x, Wl, bl, Wr, do_relu):
    BLK = 2000
    grid = (N_NODES // BLK,)
    return pl.pallas_call(
        functools.partial(_layer_body, do_relu),
        grid=grid,
        in_specs=[
            pl.BlockSpec((NC, BLK, D), lambda i: (0, i, 0)),
            pl.BlockSpec((NC, BLK, 1), lambda i: (0, i, 0)),
            pl.BlockSpec((BLK, D), lambda i: (i, 0)),
            pl.BlockSpec((D, D), lambda i: (0, 0)),
            pl.BlockSpec((1, D), lambda i: (0, 0)),
            pl.BlockSpec((D, D), lambda i: (0, 0)),
        ],
        out_specs=pl.BlockSpec((BLK, D), lambda i: (i, 0)),
        out_shape=jax.ShapeDtypeStruct((N_NODES, D), jnp.float32),
    )(aggp, degp, x, Wl, bl.reshape(1, D), Wr)


@jax.jit
def kernel(x, edge_index, W1l, b1l, W1r, W2l, b2l, W2r):
    src = edge_index[0].astype(jnp.int32)
    dst = edge_index[1].astype(jnp.int32)
    pad = E_PAD - src.shape[0]
    pi = jnp.arange(pad, dtype=jnp.int32)
    src_p = jnp.concatenate([src, pi % N_NODES]).reshape(NW, CPT, 1, CHUNK)
    dst_p = jnp.concatenate([dst, N_NODES + pi % (N_PAD - N_NODES)]
                            ).reshape(NW, CPT, 1, CHUNK)
    edges = jnp.concatenate([src_p, dst_p], axis=2)

    agg1, deg_flat = _agg_with_deg(edges, x)
    degp = deg_flat.reshape(NC, N_PAD, 1)
    h = _tc_layer(agg1, degp, x, W1l, b1l, W1r, True)
    agg2 = _agg_only(edges, h)
    return _tc_layer(agg2, degp, h, W2l, b2l, W2r, False)


# speedup vs baseline: 13.8009x; 1.0094x over previous
import functools

import jax
import jax.numpy as jnp
from jax import lax
from jax.experimental import pallas as pl
from jax.experimental.pallas import tpu as pltpu
from jax.experimental.pallas import tpu_sc as plsc

N_NODES = 10000
D = 128
NC, NS = 2, 16
NW = NC * NS
CHUNK = 128
CPT = 80
EPT = CHUNK * CPT
E_PAD = EPT * NW
N_PAD = 10240
ROWS_PER_TILE = N_PAD // NS
ZROWS = 16


def _agg_body(with_deg, *args):
    if with_deg:
        (edges_hbm, table_hbm,
         agg_out, deg_out,
         ib0, ib1, rows0, rows1, ones, zbuf, zd,
         isem0, isem1, gsem0, gsem1, ssem0, ssem1, zsem,
         dsem0, dsem1,
         agg_sh, deg_sh) = args
    else:
        (edges_hbm, table_hbm,
         agg_out,
         ib0, ib1, rows0, rows1, zbuf,
         isem0, isem1, gsem0, gsem1, ssem0, ssem1, zsem,
         agg_sh) = args
    ib = (ib0, ib1)
    isem = (isem0, isem1)
    rows = (rows0, rows1)
    gsem = (gsem0, gsem1)
    ssem = (ssem0, ssem1)
    cid = lax.axis_index("c")
    sid = lax.axis_index("s")
    wid = sid * NC + cid

    pltpu.async_copy(edges_hbm.at[wid, 0], ib0, isem0)
    pltpu.make_async_copy(edges_hbm.at[wid, 0], ib0, isem0).wait()
    pltpu.async_copy(table_hbm.at[ib0.at[0]], rows0, gsem0)

    z16 = jnp.zeros((16,), jnp.float32)

    def zero_zbuf(r, c):
        for k in range(D // 16):
            zbuf[r, pl.ds(k * 16, 16)] = z16
        return c
    lax.fori_loop(0, ZROWS, zero_zbuf, 0)

    def zero_agg(t, c):
        pltpu.async_copy(
            zbuf, agg_sh.at[pl.ds(sid * ROWS_PER_TILE + t * ZROWS, ZROWS)], zsem)
        return c
    lax.fori_loop(0, ROWS_PER_TILE // ZROWS, zero_agg, 0)

    if with_deg:
        one16 = jnp.ones((16,), jnp.float32)
        for k in range(CHUNK // 16):
            ones[pl.ds(k * 16, 16)] = one16
        for k in range(ROWS_PER_TILE // 16):
            zd[pl.ds(k * 16, 16)] = z16
        pltpu.sync_copy(zd, deg_sh.at[pl.ds(sid * ROWS_PER_TILE, ROWS_PER_TILE)])

    def zero_wait(t, c):
        pltpu.make_async_copy(
            zbuf, agg_sh.at[pl.ds(sid * ROWS_PER_TILE + t * ZROWS, ZROWS)], zsem).wait()
        return c
    lax.fori_loop(0, ROWS_PER_TILE // ZROWS, zero_wait, 0)

    plsc.subcore_barrier()

    dsem = (dsem0, dsem1) if with_deg else None

    def step(gg, c):
        for q in (0, 1):
            sc = 2 * gg + q
            ibc, ibn = ib[q], ib[1 - q]
            for b in (0, 1, 2, 3):
                t = 4 * sc + b
                rb = b % 2
                nrb = 1 - rb
                pib, prow = (ibc, 2 * b - 1) if b > 0 else (ibn, 7)

                @pl.when(t >= 1)
                def _wait_prev_scatter():
                    pltpu.make_async_copy(
                        rows[nrb], agg_sh.at[pib.at[prow]], ssem[nrb]).wait()
                    if with_deg:
                        pltpu.make_async_copy(
                            ones, deg_sh.at[pib.at[prow]], dsem[nrb]).wait()

                if b == 0:
                    @pl.when(sc + 1 < CPT // 4)
                    def _prefetch_idx():
                        pltpu.async_copy(
                            edges_hbm.at[wid, sc + 1], ibn, isem[1 - q])

                @pl.when(t + 1 < CPT)
                def _issue_next_gather():
                    if b < 3:
                        pltpu.async_copy(
                            table_hbm.at[ibc.at[2 * b + 2]], rows[nrb], gsem[nrb])
                    else:
                        pltpu.make_async_copy(
                            edges_hbm.at[wid, sc + 1], ibn, isem[1 - q]).wait()
                        pltpu.async_copy(
                            table_hbm.at[ibn.at[0]], rows[nrb], gsem[nrb])

                pltpu.make_async_copy(
                    table_hbm.at[ibc.at[2 * b]], rows[rb], gsem[rb]).wait()
                pltpu.async_copy(
                    rows[rb], agg_sh.at[ibc.at[2 * b + 1]], ssem[rb], add=True)
                if with_deg:
                    pltpu.async_copy(
                        ones, deg_sh.at[ibc.at[2 * b + 1]], dsem[rb], add=True)
        return c
    lax.fori_loop(0, CPT // 8, step, 0)

    pltpu.make_async_copy(rows[1], agg_sh.at[ib[1].at[7]], ssem[1]).wait()
    if with_deg:
        pltpu.make_async_copy(ones, deg_sh.at[ib[1].at[7]], dsem[1]).wait()

    plsc.subcore_barrier()

    pltpu.sync_copy(agg_sh.at[pl.ds(sid * ROWS_PER_TILE, ROWS_PER_TILE)],
                    agg_out.at[cid, pl.ds(sid * ROWS_PER_TILE, ROWS_PER_TILE)])
    if with_deg:
        pltpu.sync_copy(deg_sh.at[pl.ds(sid * ROWS_PER_TILE, ROWS_PER_TILE)],
                        deg_out.at[pl.ds(cid * N_PAD + sid * ROWS_PER_TILE,
                                         ROWS_PER_TILE)])


def _make_agg(with_deg):
    mesh = plsc.VectorSubcoreMesh(core_axis_name="c", subcore_axis_name="s",
                                  num_cores=NC, num_subcores=NS)
    outs = [jax.ShapeDtypeStruct((NC, N_PAD, D), jnp.float32)]
    if with_deg:
        outs.append(jax.ShapeDtypeStruct((NC * N_PAD,), jnp.float32))
    scratch = [
        pltpu.VMEM((8, CHUNK), jnp.int32),
        pltpu.VMEM((8, CHUNK), jnp.int32),
        pltpu.VMEM((CHUNK, D), jnp.float32),
        pltpu.VMEM((CHUNK, D), jnp.float32),
    ]
    if with_deg:
        scratch.append(pltpu.VMEM((CHUNK,), jnp.float32))
    scratch.append(pltpu.VMEM((ZROWS, D), jnp.float32))
    if with_deg:
        scratch.append(pltpu.VMEM((ROWS_PER_TILE,), jnp.float32))
    scratch += [pltpu.SemaphoreType.DMA] * (9 if with_deg else 7)
    scratch.append(pltpu.VMEM_SHARED((N_PAD, D), jnp.float32))
    if with_deg:
        scratch.append(pltpu.VMEM_SHARED((N_PAD,), jnp.float32))
    return pl.kernel(
        functools.partial(_agg_body, with_deg),
        out_type=tuple(outs) if with_deg else outs[0],
        mesh=mesh,
        scratch_types=scratch,
    )


_agg_with_deg = _make_agg(True)
_agg_only = _make_agg(False)


def _layer_body(do_relu, aggp_ref, degp_ref, x_ref, wl_ref, bl_ref, wr_ref, out_ref):
    deg = degp_ref[0] + degp_ref[1]
    inv = 1.0 / jnp.maximum(deg, 1.0)
    mean = (aggp_ref[0] + aggp_ref[1]) * inv
    dn = (((1,), (1,)), ((), ()))
    acc = lax.dot_general(mean, wl_ref[...], dn, preferred_element_type=jnp.float32)
    acc = acc + bl_ref[...]
    acc = acc + lax.dot_general(x_ref[...], wr_ref[...], dn,
                                preferred_element_type=jnp.float32)
    out_ref[...] = jnp.maximum(acc, 0.0) if do_relu else acc


def _tc_layer(aggp, degp, x, Wl, bl, Wr, do_relu):
    BLK = 2000
    grid = (N_NODES // BLK,)
    return pl.pallas_call(
        functools.partial(_layer_body, do_relu),
        grid=grid,
        in_specs=[
            pl.BlockSpec((NC, BLK, D), lambda i: (0, i, 0)),
            pl.BlockSpec((NC, BLK, 1), lambda i: (0, i, 0)),
            pl.BlockSpec((BLK, D), lambda i: (i, 0)),
            pl.BlockSpec((D, D), lambda i: (0, 0)),
            pl.BlockSpec((1, D), lambda i: (0, 0)),
            pl.BlockSpec((D, D), lambda i: (0, 0)),
        ],
        out_specs=pl.BlockSpec((BLK, D), lambda i: (i, 0)),
        out_shape=jax.ShapeDtypeStruct((N_NODES, D), jnp.float32),
    )(aggp, degp, x, Wl, bl.reshape(1, D), Wr)


@jax.jit
def kernel(x, edge_index, W1l, b1l, W1r, W2l, b2l, W2r):
    src = edge_index[0].astype(jnp.int32)
    dst = edge_index[1].astype(jnp.int32)
    pad = E_PAD - src.shape[0]
    pi = jnp.arange(pad, dtype=jnp.int32)
    src_p = jnp.concatenate([src, pi % N_NODES]).reshape(NW, CPT, 1, CHUNK)
    dst_p = jnp.concatenate([dst, N_NODES + pi % (N_PAD - N_NODES)]
                            ).reshape(NW, CPT, 1, CHUNK)
    edges = jnp.concatenate([src_p, dst_p], axis=2)
    edges = edges.reshape(NW, CPT // 4, 8, CHUNK)

    agg1, deg_flat = _agg_with_deg(edges, x)
    degp = deg_flat.reshape(NC, N_PAD, 1)
    h = _tc_layer(agg1, degp, x, W1l, b1l, W1r, True)
    agg2 = _agg_only(edges, h)
    return _tc_layer(agg2, degp, h, W2l, b2l, W2r, False)
